# baseline (speedup 1.0000x reference)
import functools

import jax
import jax.numpy as jnp
from jax.experimental import pallas as pl
from jax.experimental.pallas import tpu as pltpu


def _rvq_body(x_ref, cb_ref, cbb_ref, cbt_hi_ref, cbt_mid_ref, cbt_lo_ref,
              quant_ref, idx_ref, loss_ref, perp_ref,
              counts_scr, cnorm_scr, acc_scr, *, num_q, nb_code, t_len,
              total_rows, code_dim):
    b = pl.program_id(0)
    nblocks = pl.num_programs(0)

    @pl.when(b == 0)
    def _init():
        cb = cb_ref[...]
        cnorm_scr[...] = 0.5 * jnp.sum(cb * cb, axis=2, keepdims=True)
        counts_scr[...] = jnp.zeros_like(counts_scr)
        acc_scr[...] = jnp.zeros((1, 1), jnp.float32)

    xb = x_ref[0]
    r = xb
    iota = jax.lax.broadcasted_iota(jnp.int32, (nb_code, t_len), 0)
    idx_layers = []
    quant_acc = jnp.zeros_like(xb)
    for i in range(num_q):
        scores = jnp.dot(cbb_ref[i], r.astype(jnp.bfloat16),
                         preferred_element_type=jnp.float32)
        scores = scores - cnorm_scr[i]
        m = jnp.max(scores, axis=0, keepdims=True)
        idx2 = jnp.min(jnp.where(scores == m, iota, nb_code), axis=0,
                       keepdims=True)
        onehot = (iota == idx2).astype(jnp.bfloat16)
        x_d = jnp.dot(cbt_hi_ref[i], onehot, preferred_element_type=jnp.float32)
        x_d = x_d + jnp.dot(cbt_mid_ref[i], onehot,
                            preferred_element_type=jnp.float32)
        x_d = x_d + jnp.dot(cbt_lo_ref[i], onehot,
                            preferred_element_type=jnp.float32)
        ones_t = jnp.ones((t_len, 1), jnp.bfloat16)
        counts_scr[i] = counts_scr[i] + jnp.dot(
            onehot, ones_t, preferred_element_type=jnp.float32)
        diff = r - x_d
        acc_scr[...] = acc_scr[...] + jnp.sum(diff * diff, keepdims=True)
        qq = r + (x_d - r)
        quant_acc = quant_acc + qq
        r = r - qq
        idx_layers.append(idx2)

    quant_ref[0] = quant_acc
    idx_ref[0] = jnp.concatenate(idx_layers, axis=0)

    @pl.when(b == nblocks - 1)
    def _finalize():
        total = jnp.float32(total_rows)
        prob = counts_scr[...] / (total + 1e-10)
        ent = jnp.sum(prob * jnp.log(prob + 1e-7), axis=1, keepdims=True)
        perp_ref[...] = jnp.mean(jnp.exp(-ent)).reshape(1, 1)
        loss_ref[...] = acc_scr[...] / (jnp.float32(num_q) * total
                                        * jnp.float32(code_dim))


def kernel(x, codebooks):
    n, c, t = x.shape
    q, nb, cd = codebooks.shape
    cbt = jnp.transpose(codebooks, (0, 2, 1))
    cbb = codebooks.astype(jnp.bfloat16)
    mask = jnp.uint32(0xFFFF0000)
    hi_f = jax.lax.bitcast_convert_type(
        jax.lax.bitcast_convert_type(cbt, jnp.uint32) & mask, jnp.float32)
    rem = cbt - hi_f
    mid_f = jax.lax.bitcast_convert_type(
        jax.lax.bitcast_convert_type(rem, jnp.uint32) & mask, jnp.float32)
    lo_f = rem - mid_f
    cbt_hi = hi_f.astype(jnp.bfloat16)
    cbt_mid = mid_f.astype(jnp.bfloat16)
    cbt_lo = lo_f.astype(jnp.bfloat16)

    body = functools.partial(_rvq_body, num_q=q, nb_code=nb, t_len=t,
                             total_rows=n * t, code_dim=c)
    quant, idx, loss, perp = pl.pallas_call(
        body,
        grid=(n,),
        in_specs=[
            pl.BlockSpec((1, c, t), lambda b: (b, 0, 0)),
            pl.BlockSpec((q, nb, cd), lambda b: (0, 0, 0)),
            pl.BlockSpec((q, nb, cd), lambda b: (0, 0, 0)),
            pl.BlockSpec((q, cd, nb), lambda b: (0, 0, 0)),
            pl.BlockSpec((q, cd, nb), lambda b: (0, 0, 0)),
            pl.BlockSpec((q, cd, nb), lambda b: (0, 0, 0)),
        ],
        out_specs=[
            pl.BlockSpec((1, c, t), lambda b: (b, 0, 0)),
            pl.BlockSpec((1, q, t), lambda b: (b, 0, 0)),
            pl.BlockSpec((1, 1), lambda b: (0, 0)),
            pl.BlockSpec((1, 1), lambda b: (0, 0)),
        ],
        out_shape=[
            jax.ShapeDtypeStruct((n, c, t), jnp.float32),
            jax.ShapeDtypeStruct((n, q, t), jnp.int32),
            jax.ShapeDtypeStruct((1, 1), jnp.float32),
            jax.ShapeDtypeStruct((1, 1), jnp.float32),
        ],
        scratch_shapes=[
            pltpu.VMEM((q, nb, 1), jnp.float32),
            pltpu.VMEM((q, nb, 1), jnp.float32),
            pltpu.VMEM((1, 1), jnp.float32),
        ],
        compiler_params=pltpu.CompilerParams(
            dimension_semantics=("arbitrary",),
        ),
    )(x, codebooks, cbb, cbt_hi, cbt_mid, cbt_lo)

    all_indices = jnp.transpose(idx, (0, 2, 1))
    return quant, all_indices, loss[0, 0], perp[0, 0]


# speedup vs baseline: 1.2745x; 1.2745x over previous
import functools

import jax
import jax.numpy as jnp
from jax.experimental import pallas as pl
from jax.experimental.pallas import tpu as pltpu


def _rvq_body(x_ref, cb_ref, cbb_ref, cbt_hi_ref, cbt_mid_ref, cbt_lo_ref,
              quant_ref, idx_ref, loss_ref, perp_ref,
              counts_scr, cnorm_scr, acc_scr, *, num_q, nb_code, t_len,
              total_rows, code_dim):
    b = pl.program_id(0)
    nblocks = pl.num_programs(0)

    @pl.when(b == 0)
    def _init():
        cb = cb_ref[...]
        cnorm_scr[...] = 0.5 * jnp.sum(cb * cb, axis=2, keepdims=True)
        counts_scr[...] = jnp.zeros_like(counts_scr)
        acc_scr[...] = jnp.zeros((1, 1), jnp.float32)

    xb = x_ref[0]
    r = xb
    iota = jax.lax.broadcasted_iota(jnp.int32, (nb_code, t_len), 0)
    idx_layers = []
    quant_acc = jnp.zeros_like(xb)
    for i in range(num_q):
        scores = jnp.dot(cbb_ref[i], r.astype(jnp.bfloat16),
                         preferred_element_type=jnp.float32)
        scores = scores - cnorm_scr[i]
        idx2 = jnp.argmax(scores, axis=0, keepdims=True).astype(jnp.int32)
        onehot = (iota == idx2).astype(jnp.bfloat16)
        dn = (((0,), (0,)), ((), ()))
        x_d = jax.lax.dot_general(cbt_hi_ref[i], onehot, dn,
                                  preferred_element_type=jnp.float32)
        x_d = x_d + jax.lax.dot_general(cbt_mid_ref[i], onehot, dn,
                                        preferred_element_type=jnp.float32)
        x_d = x_d + jax.lax.dot_general(cbt_lo_ref[i], onehot, dn,
                                        preferred_element_type=jnp.float32)
        counts_scr[i] = counts_scr[i] + jnp.sum(
            onehot.astype(jnp.float32), axis=1, keepdims=True)
        diff = r - x_d
        acc_scr[...] = acc_scr[...] + jnp.sum(diff * diff, keepdims=True)
        qq = r + (x_d - r)
        quant_acc = quant_acc + qq
        r = r - qq
        idx_layers.append(idx2)

    quant_ref[0] = quant_acc
    idx_ref[0] = jnp.concatenate(idx_layers, axis=0)

    @pl.when(b == nblocks - 1)
    def _finalize():
        total = jnp.float32(total_rows)
        prob = counts_scr[...] / (total + 1e-10)
        ent = jnp.sum(prob * jnp.log(prob + 1e-7), axis=1, keepdims=True)
        perp_ref[...] = jnp.mean(jnp.exp(-ent)).reshape(1, 1)
        loss_ref[...] = acc_scr[...] / (jnp.float32(num_q) * total
                                        * jnp.float32(code_dim))


def kernel(x, codebooks):
    n, c, t = x.shape
    q, nb, cd = codebooks.shape
    cbb = codebooks.astype(jnp.bfloat16)
    mask = jnp.uint32(0xFFFF0000)
    hi_f = jax.lax.bitcast_convert_type(
        jax.lax.bitcast_convert_type(codebooks, jnp.uint32) & mask,
        jnp.float32)
    rem = codebooks - hi_f
    mid_f = jax.lax.bitcast_convert_type(
        jax.lax.bitcast_convert_type(rem, jnp.uint32) & mask, jnp.float32)
    lo_f = rem - mid_f
    cbt_hi = hi_f.astype(jnp.bfloat16)
    cbt_mid = mid_f.astype(jnp.bfloat16)
    cbt_lo = lo_f.astype(jnp.bfloat16)

    body = functools.partial(_rvq_body, num_q=q, nb_code=nb, t_len=t,
                             total_rows=n * t, code_dim=c)
    quant, idx, loss, perp = pl.pallas_call(
        body,
        grid=(n,),
        in_specs=[
            pl.BlockSpec((1, c, t), lambda b: (b, 0, 0)),
            pl.BlockSpec((q, nb, cd), lambda b: (0, 0, 0)),
            pl.BlockSpec((q, nb, cd), lambda b: (0, 0, 0)),
            pl.BlockSpec((q, nb, cd), lambda b: (0, 0, 0)),
            pl.BlockSpec((q, nb, cd), lambda b: (0, 0, 0)),
            pl.BlockSpec((q, nb, cd), lambda b: (0, 0, 0)),
        ],
        out_specs=[
            pl.BlockSpec((1, c, t), lambda b: (b, 0, 0)),
            pl.BlockSpec((1, q, t), lambda b: (b, 0, 0)),
            pl.BlockSpec((1, 1), lambda b: (0, 0)),
            pl.BlockSpec((1, 1), lambda b: (0, 0)),
        ],
        out_shape=[
            jax.ShapeDtypeStruct((n, c, t), jnp.float32),
            jax.ShapeDtypeStruct((n, q, t), jnp.int32),
            jax.ShapeDtypeStruct((1, 1), jnp.float32),
            jax.ShapeDtypeStruct((1, 1), jnp.float32),
        ],
        scratch_shapes=[
            pltpu.VMEM((q, nb, 1), jnp.float32),
            pltpu.VMEM((q, nb, 1), jnp.float32),
            pltpu.VMEM((1, 1), jnp.float32),
        ],
        compiler_params=pltpu.CompilerParams(
            dimension_semantics=("arbitrary",),
        ),
    )(x, codebooks, cbb, cbt_hi, cbt_mid, cbt_lo)

    all_indices = jnp.transpose(idx, (0, 2, 1))
    return quant, all_indices, loss[0, 0], perp[0, 0]


# speedup vs baseline: 1.4330x; 1.1244x over previous
import functools

import jax
import jax.numpy as jnp
from jax.experimental import pallas as pl
from jax.experimental.pallas import tpu as pltpu


def _rvq_body(x_ref, cb_ref, cbb_ref, cbt_hi_ref, cbt_mid_ref, cbt_lo_ref,
              quant_ref, idx_ref, loss_ref, perp_ref,
              counts_scr, cnorm_scr, acc_scr, *, num_q, nb_code, t_len,
              total_rows, code_dim, blk_n):
    b = pl.program_id(0)
    nblocks = pl.num_programs(0)

    @pl.when(b == 0)
    def _init():
        cb = cb_ref[...]
        cnorm_scr[...] = 0.5 * jnp.sum(cb * cb, axis=2, keepdims=True)
        counts_scr[...] = jnp.zeros_like(counts_scr)
        acc_scr[...] = jnp.zeros((1, 1), jnp.float32)

    xb = jnp.concatenate([x_ref[j] for j in range(blk_n)], axis=1)
    r = xb
    iota = jax.lax.broadcasted_iota(jnp.int32, (nb_code, blk_n * t_len), 0)
    idx_layers = []
    quant_acc = jnp.zeros_like(xb)
    for i in range(num_q):
        scores = jnp.dot(cbb_ref[i], r.astype(jnp.bfloat16),
                         preferred_element_type=jnp.float32)
        scores = scores - cnorm_scr[i]
        idx2 = jnp.argmax(scores, axis=0, keepdims=True).astype(jnp.int32)
        onehot = (iota == idx2).astype(jnp.bfloat16)
        dn = (((0,), (0,)), ((), ()))
        x_d = jax.lax.dot_general(cbt_hi_ref[i], onehot, dn,
                                  preferred_element_type=jnp.float32)
        x_d = x_d + jax.lax.dot_general(cbt_mid_ref[i], onehot, dn,
                                        preferred_element_type=jnp.float32)
        x_d = x_d + jax.lax.dot_general(cbt_lo_ref[i], onehot, dn,
                                        preferred_element_type=jnp.float32)
        counts_scr[i] = counts_scr[i] + jnp.sum(
            onehot.astype(jnp.float32), axis=1, keepdims=True)
        diff = r - x_d
        acc_scr[...] = acc_scr[...] + jnp.sum(diff * diff, keepdims=True)
        qq = r + (x_d - r)
        quant_acc = quant_acc + qq
        r = r - qq
        idx_layers.append(idx2)

    idx_all = jnp.concatenate(idx_layers, axis=0)
    for j in range(blk_n):
        quant_ref[j] = quant_acc[:, j * t_len:(j + 1) * t_len]
        idx_ref[j] = idx_all[:, j * t_len:(j + 1) * t_len]

    @pl.when(b == nblocks - 1)
    def _finalize():
        total = jnp.float32(total_rows)
        prob = counts_scr[...] / (total + 1e-10)
        ent = jnp.sum(prob * jnp.log(prob + 1e-7), axis=1, keepdims=True)
        perp_ref[...] = jnp.mean(jnp.exp(-ent)).reshape(1, 1)
        loss_ref[...] = acc_scr[...] / (jnp.float32(num_q) * total
                                        * jnp.float32(code_dim))


def kernel(x, codebooks):
    n, c, t = x.shape
    q, nb, cd = codebooks.shape
    cbb = codebooks.astype(jnp.bfloat16)
    mask = jnp.uint32(0xFFFF0000)
    hi_f = jax.lax.bitcast_convert_type(
        jax.lax.bitcast_convert_type(codebooks, jnp.uint32) & mask,
        jnp.float32)
    rem = codebooks - hi_f
    mid_f = jax.lax.bitcast_convert_type(
        jax.lax.bitcast_convert_type(rem, jnp.uint32) & mask, jnp.float32)
    lo_f = rem - mid_f
    cbt_hi = hi_f.astype(jnp.bfloat16)
    cbt_mid = mid_f.astype(jnp.bfloat16)
    cbt_lo = lo_f.astype(jnp.bfloat16)

    blk_n = 2 if n % 2 == 0 else 1
    body = functools.partial(_rvq_body, num_q=q, nb_code=nb, t_len=t,
                             total_rows=n * t, code_dim=c, blk_n=blk_n)
    quant, idx, loss, perp = pl.pallas_call(
        body,
        grid=(n // blk_n,),
        in_specs=[
            pl.BlockSpec((blk_n, c, t), lambda b: (b, 0, 0)),
            pl.BlockSpec((q, nb, cd), lambda b: (0, 0, 0)),
            pl.BlockSpec((q, nb, cd), lambda b: (0, 0, 0)),
            pl.BlockSpec((q, nb, cd), lambda b: (0, 0, 0)),
            pl.BlockSpec((q, nb, cd), lambda b: (0, 0, 0)),
            pl.BlockSpec((q, nb, cd), lambda b: (0, 0, 0)),
        ],
        out_specs=[
            pl.BlockSpec((blk_n, c, t), lambda b: (b, 0, 0)),
            pl.BlockSpec((blk_n, q, t), lambda b: (b, 0, 0)),
            pl.BlockSpec((1, 1), lambda b: (0, 0)),
            pl.BlockSpec((1, 1), lambda b: (0, 0)),
        ],
        out_shape=[
            jax.ShapeDtypeStruct((n, c, t), jnp.float32),
            jax.ShapeDtypeStruct((n, q, t), jnp.int32),
            jax.ShapeDtypeStruct((1, 1), jnp.float32),
            jax.ShapeDtypeStruct((1, 1), jnp.float32),
        ],
        scratch_shapes=[
            pltpu.VMEM((q, nb, 1), jnp.float32),
            pltpu.VMEM((q, nb, 1), jnp.float32),
            pltpu.VMEM((1, 1), jnp.float32),
        ],
        compiler_params=pltpu.CompilerParams(
            dimension_semantics=("arbitrary",),
        ),
    )(x, codebooks, cbb, cbt_hi, cbt_mid, cbt_lo)

    all_indices = jnp.transpose(idx, (0, 2, 1))
    return quant, all_indices, loss[0, 0], perp[0, 0]
